# Initial kernel scaffold; baseline (speedup 1.0000x reference)
import jax
import jax.numpy as jnp
from jax.experimental import pallas as pl


def kernel(x_nchw, conv1_w, conv1_b, conv2_w, conv2_b, conv3_w, conv3_b, fc1_w, fc1_b, fc2_w, fc2_b, fc3_w, fc3_b):
    raise NotImplementedError("write your pallas kernel here")


# speedup vs baseline: 7.0028x; 7.0028x over previous
import functools

import jax
import jax.numpy as jnp
from jax.experimental import pallas as pl
from jax.experimental.pallas import tpu as pltpu

_VMEM_LIMIT = 48 * 1024 * 1024


def _pool_bias_relu(acc, rb, wp, cout, b_ref):
    a = acc.reshape(2 * rb * wp, 2, cout)
    m = jnp.maximum(a[:, 0, :], a[:, 1, :])
    m = m.reshape(rb, 2, wp, cout)
    m = jnp.maximum(m[:, 0], m[:, 1])
    return jnp.maximum(m + b_ref[...].reshape(1, 1, cout), 0.0)


def _c1_kernel(x_ref, w_ref, b_ref, o_ref, *, rb):
    row0 = pl.program_id(1) * 2 * rb
    pieces = [x_ref[0, pl.ds(row0 + di, 2 * rb), :, :] for di in range(5)]
    xm = jnp.concatenate(pieces, axis=-1).reshape(2 * rb * 220, 75)
    acc = jnp.dot(xm, w_ref[...], preferred_element_type=jnp.float32)
    out = _pool_bias_relu(acc, rb, 110, 64, b_ref)
    o_ref[...] = out.astype(o_ref.dtype).reshape(1, rb, 110, 64)


def _c3x3_kernel(x_ref, w_ref, b_ref, o_ref, *, rb):
    _, _, w_in, cin = x_ref.shape
    cout = o_ref.shape[-1]
    wo = w_in - 2
    wp = wo // 2
    row0 = pl.program_id(1) * 2 * rb
    pieces = [x_ref[0, pl.ds(row0 + di, 2 * rb), pl.ds(dj, wo), :]
              for di in range(3) for dj in range(3)]
    xm = jnp.concatenate(pieces, axis=-1).reshape(2 * rb * wo, 9 * cin)
    acc = jnp.dot(xm, w_ref[...], preferred_element_type=jnp.float32)
    out = _pool_bias_relu(acc, rb, wp, cout, b_ref)
    o_ref[...] = out.astype(o_ref.dtype).reshape(1, rb, wp, cout)


def _conv1_pool(xp, w, b, *, rb):
    n = xp.shape[0]
    return pl.pallas_call(
        functools.partial(_c1_kernel, rb=rb),
        out_shape=jax.ShapeDtypeStruct((n, 110, 110, 64), jnp.bfloat16),
        grid_spec=pltpu.PrefetchScalarGridSpec(
            num_scalar_prefetch=0,
            grid=(n, 110 // rb),
            in_specs=[
                pl.BlockSpec((1, 224, 220, 15), lambda i, r: (i, 0, 0, 0)),
                pl.BlockSpec((75, 64), lambda i, r: (0, 0)),
                pl.BlockSpec((1, 64), lambda i, r: (0, 0)),
            ],
            out_specs=pl.BlockSpec((1, rb, 110, 64), lambda i, r: (i, r, 0, 0)),
        ),
        compiler_params=pltpu.CompilerParams(
            dimension_semantics=("parallel", "arbitrary"),
            vmem_limit_bytes=_VMEM_LIMIT),
    )(xp, w, b)


def _conv3x3_pool(x, w, b, *, rb):
    n, h, w_in, cin = x.shape
    cout = w.shape[-1]
    wp = (w_in - 2) // 2
    hp = (h - 2) // 2
    w_r = w.astype(jnp.bfloat16).reshape(9 * cin, cout)
    b_r = b.astype(jnp.float32).reshape(1, cout)
    return pl.pallas_call(
        functools.partial(_c3x3_kernel, rb=rb),
        out_shape=jax.ShapeDtypeStruct((n, hp, wp, cout), jnp.bfloat16),
        grid_spec=pltpu.PrefetchScalarGridSpec(
            num_scalar_prefetch=0,
            grid=(n, hp // rb),
            in_specs=[
                pl.BlockSpec((1, h, w_in, cin), lambda i, r: (i, 0, 0, 0)),
                pl.BlockSpec((9 * cin, cout), lambda i, r: (0, 0)),
                pl.BlockSpec((1, cout), lambda i, r: (0, 0)),
            ],
            out_specs=pl.BlockSpec((1, rb, wp, cout), lambda i, r: (i, r, 0, 0)),
        ),
        compiler_params=pltpu.CompilerParams(
            dimension_semantics=("parallel", "arbitrary"),
            vmem_limit_bytes=_VMEM_LIMIT),
    )(x, w_r, b_r)


def _fc_head_kernel(x_ref, w1_ref, b1_ref, w2_ref, b2_ref, w3_ref, b3_ref,
                    o_ref):
    h = jnp.dot(x_ref[...].astype(jnp.float32), w1_ref[...],
                preferred_element_type=jnp.float32) + b1_ref[...]
    h = jnp.maximum(h, 0.0)
    h = jnp.dot(h, w2_ref[...], preferred_element_type=jnp.float32) + b2_ref[...]
    h = jnp.maximum(h, 0.0)
    o = jnp.dot(h, w3_ref[...], preferred_element_type=jnp.float32) + b3_ref[...]
    o_ref[...] = o


def kernel(x_nchw, conv1_w, conv1_b, conv2_w, conv2_b, conv3_w, conv3_b,
           fc1_w, fc1_b, fc2_w, fc2_b, fc3_w, fc3_b):
    n = x_nchw.shape[0]

    xt = jnp.transpose(x_nchw, (0, 2, 3, 1)).astype(jnp.bfloat16)
    xp = jnp.concatenate([xt[:, :, dj:dj + 220, :] for dj in range(5)],
                         axis=-1)
    w1 = conv1_w.astype(jnp.bfloat16).reshape(75, 64)
    b1 = conv1_b.astype(jnp.float32).reshape(1, 64)

    x = _conv1_pool(xp, w1, b1, rb=55)
    x = _conv3x3_pool(x, conv2_w, conv2_b, rb=27)
    x = _conv3x3_pool(x, conv3_w, conv3_b, rb=26)

    w1p = fc1_w.reshape(16, 26, 26, 120).transpose(1, 2, 0, 3).reshape(10816, 120)
    xf = x.reshape(n, 10816)

    return pl.pallas_call(
        _fc_head_kernel,
        out_shape=jax.ShapeDtypeStruct((n, fc3_w.shape[1]), jnp.float32),
        compiler_params=pltpu.CompilerParams(vmem_limit_bytes=_VMEM_LIMIT),
    )(xf, w1p, fc1_b.reshape(1, -1),
      fc2_w, fc2_b.reshape(1, -1),
      fc3_w, fc3_b.reshape(1, -1))
